# baseline (speedup 1.0000x reference)
import functools

import jax
import jax.numpy as jnp
from jax.experimental import pallas as pl
from jax.experimental.pallas import tpu as pltpu


def _l2norm(x, eps=1e-12):
    d = jnp.sqrt(jnp.sum(x * x, axis=(2, 3), keepdims=True))
    return x / jnp.maximum(d, eps)


def _stats(x):
    rgb, ir = x[0], x[1]
    rgb_cap = jnp.mean(rgb, axis=1, keepdims=True)
    rgb_cmp = jnp.max(rgb, axis=1, keepdims=True)
    ir_cap = jnp.mean(ir, axis=1, keepdims=True)
    ir_cmp = jnp.max(ir, axis=1, keepdims=True)
    x1_cp = jnp.concatenate([rgb_cap, rgb_cmp], axis=1)
    x2_cp = jnp.concatenate([ir_cap, ir_cmp], axis=1)
    cp = x1_cp + x2_cp
    sa = jnp.maximum(cp[:, ::2, :, :], cp[:, 1::2, :, :])
    sa_sig = jax.nn.sigmoid(sa)
    sa_norm = _l2norm(sa_sig)
    sim_rgb = jnp.sum(sa_norm * _l2norm(rgb), axis=(2, 3))
    sim_ir = jnp.sum(sa_norm * _l2norm(ir), axis=(2, 3))
    return sa, sim_rgb, sim_ir


def _ksort(srow_ref, scol_ref, sims_ref, src_ref, pmeta_ref, idxm_ref):
    C = srow_ref.shape[3]
    s = pl.program_id(0)
    srow = srow_ref[0, 0]
    scol = scol_ref[0, 0]
    iota_row = jax.lax.broadcasted_iota(jnp.int32, (1, C), 1)
    iota_col = jax.lax.broadcasted_iota(jnp.int32, (C, 1), 0)
    before = (srow < scol) | ((srow == scol) & (iota_row < iota_col))
    rank = jnp.sum(jnp.where(before, 1.0, 0.0), axis=1,
                   keepdims=True).astype(jnp.int32)

    allsims = sims_ref[...]
    cnt = jnp.sum(jnp.where(allsims > 0, 1.0, 0.0), axis=3)
    k0 = jnp.max(cnt[0]).astype(jnp.int32)
    k1 = jnp.max(cnt[1]).astype(jnp.int32)
    is0 = s == 0
    act = jnp.where(is0, (k1 > k0) & (k0 > 0), (k0 > k1) & (k1 > 0))
    kk = jnp.where(is0, k0, k1)

    pos_act = jnp.where(rank < kk, rank,
                        jnp.where(rank == C - 1, kk, rank + 1))
    pos = jnp.where(act, pos_act, rank)

    mat = pos == iota_row
    src = jnp.sum(jnp.where(mat, iota_col.astype(jnp.float32), 0.0),
                  axis=0, keepdims=True).astype(jnp.int32)
    src_ref[0, 0] = src

    ppos = jnp.where(act, kk, 0)
    acti = act.astype(jnp.int32)
    lanes = jax.lax.broadcasted_iota(jnp.int32, (1, 128), 1)
    pmeta_ref[0, 0] = jnp.where(lanes == 0, ppos,
                                jnp.where(lanes == 1, acti, 0))
    idxm = jnp.sum(jnp.where(rank == 0, iota_col, 0))
    idxm_ref[0, 0] = jnp.zeros((1, 128), jnp.int32) + idxm


def _kperm(src_ref, pmeta_ref, idxm_ref, x_ref, rowa_ref, rowb_ref, sig_ref,
           out_ref, *, stream):
    C = x_ref.shape[2]
    b = pl.program_id(0)
    sig = sig_ref[0, 0]

    def body(p, carry):
        c = src_ref[stream, b, p]
        out_ref[0, p] = x_ref[0, 0, c] * sig
        return carry

    jax.lax.fori_loop(0, C, body, 0, unroll=8)

    ra = rowa_ref[0, 0, 0]
    rb = rowb_ref[0, 0, 0]
    own = ra if stream == 0 else rb
    act = pmeta_ref[stream, b, 0, 1] != 0
    prow = jnp.where(act, jnp.maximum(ra, rb), own)
    pp = pmeta_ref[stream, b, 0, 0]
    out_ref[0, pp] = prow * sig


def kernel(x):
    S, B, C, H, W = x.shape
    f32 = jnp.float32

    sa, sim_rgb, sim_ir = _stats(x)
    sa_sig = jax.nn.sigmoid(sa)
    sims = jnp.stack([sim_rgb, sim_ir]).reshape(S, B, 1, C)
    sims_col = sims.reshape(S, B, C, 1)

    src, pmeta, idxm = pl.pallas_call(
        _ksort,
        grid=(S, B),
        in_specs=[
            pl.BlockSpec((1, 1, 1, C), lambda s, b: (s, b, 0, 0)),
            pl.BlockSpec((1, 1, C, 1), lambda s, b: (s, b, 0, 0)),
            pl.BlockSpec((S, B, 1, C), lambda s, b: (0, 0, 0, 0)),
        ],
        out_specs=[pl.BlockSpec((1, 1, 1, C), lambda s, b: (s, b, 0, 0)),
                   pl.BlockSpec((1, 1, 1, 128), lambda s, b: (s, b, 0, 0)),
                   pl.BlockSpec((1, 1, 1, 128), lambda s, b: (s, b, 0, 0))],
        out_shape=[jax.ShapeDtypeStruct((S, B, 1, C), jnp.int32),
                   jax.ShapeDtypeStruct((S, B, 1, 128), jnp.int32),
                   jax.ShapeDtypeStruct((S, B, 1, 128), jnp.int32)],
    )(sims, sims_col, sims)
    src = src.reshape(S, B, C)

    outs = []
    for stream in (0, 1):
        grid_spec = pltpu.PrefetchScalarGridSpec(
            num_scalar_prefetch=3,
            grid=(B,),
            in_specs=[
                pl.BlockSpec((1, 1, C, H, W),
                             lambda b, sr, pm, im, st=stream: (st, b, 0, 0, 0)),
                pl.BlockSpec((1, 1, 1, H, W),
                             lambda b, sr, pm, im: (0, b, im[0, b, 0, 0], 0, 0)),
                pl.BlockSpec((1, 1, 1, H, W),
                             lambda b, sr, pm, im: (1, b, im[1, b, 0, 0], 0, 0)),
                pl.BlockSpec((1, 1, H, W), lambda b, sr, pm, im: (b, 0, 0, 0)),
            ],
            out_specs=pl.BlockSpec((1, C, H, W),
                                   lambda b, sr, pm, im: (b, 0, 0, 0)),
        )
        out = pl.pallas_call(
            functools.partial(_kperm, stream=stream),
            grid_spec=grid_spec,
            out_shape=jax.ShapeDtypeStruct((B, C, H, W), f32),
        )(src, pmeta, idxm, x, x, x, sa_sig)
        outs.append(out)

    return outs[0], outs[1]


# speedup vs baseline: 1.4871x; 1.4871x over previous
import functools

import jax
import jax.numpy as jnp
from jax.experimental import pallas as pl
from jax.experimental.pallas import tpu as pltpu


def _l2norm(x, eps=1e-12):
    d = jnp.sqrt(jnp.sum(x * x, axis=(2, 3), keepdims=True))
    return x / jnp.maximum(d, eps)


def _stats(x):
    rgb, ir = x[0], x[1]
    rgb_cap = jnp.mean(rgb, axis=1, keepdims=True)
    rgb_cmp = jnp.max(rgb, axis=1, keepdims=True)
    ir_cap = jnp.mean(ir, axis=1, keepdims=True)
    ir_cmp = jnp.max(ir, axis=1, keepdims=True)
    x1_cp = jnp.concatenate([rgb_cap, rgb_cmp], axis=1)
    x2_cp = jnp.concatenate([ir_cap, ir_cmp], axis=1)
    cp = x1_cp + x2_cp
    sa = jnp.maximum(cp[:, ::2, :, :], cp[:, 1::2, :, :])
    sa_sig = jax.nn.sigmoid(sa)
    sa_norm = _l2norm(sa_sig)
    sim_rgb = jnp.sum(sa_norm * _l2norm(rgb), axis=(2, 3))
    sim_ir = jnp.sum(sa_norm * _l2norm(ir), axis=(2, 3))
    return sa, sim_rgb, sim_ir


def _insert_pos(rank, kk, act, C):
    pos_act = jnp.where(rank < kk, rank,
                        jnp.where(rank == C - 1, kk, rank + 1))
    return jnp.where(act, pos_act, rank)


def _ksort(srow_ref, scol_ref, sims_ref, posr_ref, srcr_ref, pmeta_ref,
           idxm_ref):
    C = srow_ref.shape[3]
    s = pl.program_id(0)
    srow = srow_ref[0, 0]
    scol = scol_ref[0, 0]
    iota_row = jax.lax.broadcasted_iota(jnp.int32, (1, C), 1)
    iota_col = jax.lax.broadcasted_iota(jnp.int32, (C, 1), 0)
    beforeR = (scol < srow) | ((scol == srow) & (iota_col < iota_row))
    rank_row = jnp.sum(jnp.where(beforeR, 1.0, 0.0), axis=0,
                       keepdims=True).astype(jnp.int32)
    beforeC = (srow < scol) | ((srow == scol) & (iota_row < iota_col))
    rank_col = jnp.sum(jnp.where(beforeC, 1.0, 0.0), axis=1,
                       keepdims=True).astype(jnp.int32)

    allsims = sims_ref[...]
    cnt = jnp.sum(jnp.where(allsims > 0, 1.0, 0.0), axis=3)
    k0 = jnp.max(cnt[0]).astype(jnp.int32)
    k1 = jnp.max(cnt[1]).astype(jnp.int32)
    is0 = s == 0
    act = jnp.where(is0, (k1 > k0) & (k0 > 0), (k0 > k1) & (k1 > 0))
    kk = jnp.where(is0, k0, k1)

    pos_row = _insert_pos(rank_row, kk, act, C)
    pos_col = _insert_pos(rank_col, kk, act, C)
    posr_ref[0, 0] = pos_row

    mat = pos_col == iota_row
    srcr_ref[0, 0] = jnp.sum(
        jnp.where(mat, iota_col.astype(jnp.float32), 0.0),
        axis=0, keepdims=True).astype(jnp.int32)

    ppos = jnp.where(act, kk, 0)
    acti = act.astype(jnp.int32)
    lanes = jax.lax.broadcasted_iota(jnp.int32, (1, 128), 1)
    pmeta_ref[0, 0] = jnp.where(lanes == 0, ppos,
                                jnp.where(lanes == 1, acti, 0))
    idxm = jnp.sum(jnp.where(rank_col == 0, iota_col, 0))
    idxm_ref[0, 0] = jnp.zeros((1, 128), jnp.int32) + idxm


def _kperm(pmeta_ref, idxm_ref, x_ref, rowa_ref, rowb_ref, pos_ref, sig_ref,
           out_ref, *, stream):
    C = x_ref.shape[2]
    b = pl.program_id(0)
    xb = x_ref[0, 0]
    posr = pos_ref[0, 0]
    iota_col = jax.lax.broadcasted_iota(jnp.int32, (C, 1), 0)
    P = (iota_col == posr).astype(jnp.float32)
    out = jax.lax.dot_general(
        P, xb, (((1,), (0,)), ((), ())),
        preferred_element_type=jnp.float32)
    ra = rowa_ref[0, 0, 0]
    rb = rowb_ref[0, 0, 0]
    own = ra if stream == 0 else rb
    act = pmeta_ref[stream, b, 0, 1] != 0
    prow = jnp.where(act, jnp.maximum(ra, rb), own)
    pp = pmeta_ref[stream, b, 0, 0]
    out = jnp.where(iota_col == pp, prow, out)
    out_ref[0] = out * sig_ref[0, 0]


def kernel(x):
    S, B, C, H, W = x.shape
    HW = H * W
    f32 = jnp.float32

    sa, sim_rgb, sim_ir = _stats(x)
    sa_sig = jax.nn.sigmoid(sa)
    sims = jnp.stack([sim_rgb, sim_ir]).reshape(S, B, 1, C)
    sims_col = sims.reshape(S, B, C, 1)
    sig_arr = sa_sig.reshape(B, 1, HW)
    xr = x.reshape(S, B, C, HW)
    xr5 = xr.reshape(S, B, C, 1, HW)

    posr, srcr, pmeta, idxm = pl.pallas_call(
        _ksort,
        grid=(S, B),
        in_specs=[
            pl.BlockSpec((1, 1, 1, C), lambda s, b: (s, b, 0, 0)),
            pl.BlockSpec((1, 1, C, 1), lambda s, b: (s, b, 0, 0)),
            pl.BlockSpec((S, B, 1, C), lambda s, b: (0, 0, 0, 0)),
        ],
        out_specs=[pl.BlockSpec((1, 1, 1, C), lambda s, b: (s, b, 0, 0)),
                   pl.BlockSpec((1, 1, 1, C), lambda s, b: (s, b, 0, 0)),
                   pl.BlockSpec((1, 1, 1, 128), lambda s, b: (s, b, 0, 0)),
                   pl.BlockSpec((1, 1, 1, 128), lambda s, b: (s, b, 0, 0))],
        out_shape=[jax.ShapeDtypeStruct((S, B, 1, C), jnp.int32),
                   jax.ShapeDtypeStruct((S, B, 1, C), jnp.int32),
                   jax.ShapeDtypeStruct((S, B, 1, 128), jnp.int32),
                   jax.ShapeDtypeStruct((S, B, 1, 128), jnp.int32)],
    )(sims, sims_col, sims)

    outs = []
    for stream in (0, 1):
        grid_spec = pltpu.PrefetchScalarGridSpec(
            num_scalar_prefetch=2,
            grid=(B,),
            in_specs=[
                pl.BlockSpec((1, 1, C, HW),
                             lambda b, pm, im, st=stream: (st, b, 0, 0)),
                pl.BlockSpec((1, 1, 1, 1, HW),
                             lambda b, pm, im: (0, b, im[0, b, 0, 0], 0, 0)),
                pl.BlockSpec((1, 1, 1, 1, HW),
                             lambda b, pm, im: (1, b, im[1, b, 0, 0], 0, 0)),
                pl.BlockSpec((1, 1, 1, C),
                             lambda b, pm, im, st=stream: (st, b, 0, 0)),
                pl.BlockSpec((1, 1, HW), lambda b, pm, im: (b, 0, 0)),
            ],
            out_specs=pl.BlockSpec((1, C, HW), lambda b, pm, im: (b, 0, 0)),
        )
        out = pl.pallas_call(
            functools.partial(_kperm, stream=stream),
            grid_spec=grid_spec,
            out_shape=jax.ShapeDtypeStruct((B, C, HW), f32),
        )(pmeta, idxm, xr, xr5, xr5, posr, sig_arr)
        outs.append(out.reshape(B, C, H, W))

    return outs[0], outs[1]


# speedup vs baseline: 2.4714x; 1.6619x over previous
import jax
import jax.numpy as jnp
from jax.experimental import pallas as pl
from jax.experimental.pallas import tpu as pltpu


def _l2norm(x, eps=1e-12):
    d = jnp.sqrt(jnp.sum(x * x, axis=(2, 3), keepdims=True))
    return x / jnp.maximum(d, eps)


def _stats(x):
    rgb, ir = x[0], x[1]
    rgb_cap = jnp.mean(rgb, axis=1, keepdims=True)
    rgb_cmp = jnp.max(rgb, axis=1, keepdims=True)
    ir_cap = jnp.mean(ir, axis=1, keepdims=True)
    ir_cmp = jnp.max(ir, axis=1, keepdims=True)
    x1_cp = jnp.concatenate([rgb_cap, rgb_cmp], axis=1)
    x2_cp = jnp.concatenate([ir_cap, ir_cmp], axis=1)
    cp = x1_cp + x2_cp
    sa = jnp.maximum(cp[:, ::2, :, :], cp[:, 1::2, :, :])
    sa_sig = jax.nn.sigmoid(sa)
    sa_norm = _l2norm(sa_sig)
    sim_rgb = jnp.sum(sa_norm * _l2norm(rgb), axis=(2, 3))
    sim_ir = jnp.sum(sa_norm * _l2norm(ir), axis=(2, 3))
    return sa, sim_rgb, sim_ir


def _kmain(srow_ref, scol_ref, sims_ref, x_ref, sig_ref, out0_ref, out1_ref,
           minrow_ref):
    C = x_ref.shape[2]
    s = pl.program_id(1)
    srow = srow_ref[0, 0]
    scol = scol_ref[0, 0]
    iota_row = jax.lax.broadcasted_iota(jnp.int32, (1, C), 1)
    iota_col = jax.lax.broadcasted_iota(jnp.int32, (C, 1), 0)
    before = (scol < srow) | ((scol == srow) & (iota_col < iota_row))
    rank = jnp.sum(jnp.where(before, 1.0, 0.0), axis=0,
                   keepdims=True).astype(jnp.int32)

    allsims = sims_ref[...]
    cnt = jnp.sum(jnp.where(allsims > 0, 1.0, 0.0), axis=3)
    k0 = jnp.max(cnt[0]).astype(jnp.int32)
    k1 = jnp.max(cnt[1]).astype(jnp.int32)
    act0 = (k1 > k0) & (k0 > 0)
    act1 = (k0 > k1) & (k1 > 0)
    is0 = s == 0
    act = jnp.where(is0, act0, act1)
    kk = jnp.where(is0, k0, k1)

    pos_act = jnp.where(rank < kk, rank,
                        jnp.where(rank == C - 1, kk, rank + 1))
    pos = jnp.where(act, pos_act, rank)

    xb = x_ref[0, 0]
    sig = sig_ref[0, 0]
    P = (iota_col == pos).astype(jnp.float32)
    out = jax.lax.dot_general(
        P, xb, (((1,), (0,)), ((), ())),
        preferred_element_type=jnp.float32) * sig

    ohmin = (rank == 0).astype(jnp.float32)
    rowmin = jax.lax.dot_general(
        ohmin, xb, (((1,), (0,)), ((), ())),
        preferred_element_type=jnp.float32)

    @pl.when(is0)
    def _():
        out0_ref[0] = out
        minrow_ref[...] = rowmin

    @pl.when(jnp.logical_not(is0))
    def _():
        ef = jnp.maximum(minrow_ref[...], rowmin)
        out1_ref[0] = jnp.where(act1 & (iota_col == k1), ef * sig, out)

        @pl.when(act0)
        def _():
            out0_ref[0] = jnp.where(iota_col == k0, ef * sig, out0_ref[0])


def kernel(x):
    S, B, C, H, W = x.shape
    HW = H * W
    f32 = jnp.float32

    sa, sim_rgb, sim_ir = _stats(x)
    sa_sig = jax.nn.sigmoid(sa)
    sims = jnp.stack([sim_rgb, sim_ir]).reshape(S, B, 1, C)
    sims_col = sims.reshape(S, B, C, 1)
    sig_arr = sa_sig.reshape(B, 1, HW)
    xr = x.reshape(S, B, C, HW)

    out0, out1 = pl.pallas_call(
        _kmain,
        grid=(B, S),
        in_specs=[
            pl.BlockSpec((1, 1, 1, C), lambda b, s: (s, b, 0, 0)),
            pl.BlockSpec((1, 1, C, 1), lambda b, s: (s, b, 0, 0)),
            pl.BlockSpec((S, B, 1, C), lambda b, s: (0, 0, 0, 0)),
            pl.BlockSpec((1, 1, C, HW), lambda b, s: (s, b, 0, 0)),
            pl.BlockSpec((1, 1, HW), lambda b, s: (b, 0, 0)),
        ],
        out_specs=[pl.BlockSpec((1, C, HW), lambda b, s: (b, 0, 0)),
                   pl.BlockSpec((1, C, HW), lambda b, s: (b, 0, 0))],
        out_shape=[jax.ShapeDtypeStruct((B, C, HW), f32),
                   jax.ShapeDtypeStruct((B, C, HW), f32)],
        scratch_shapes=[pltpu.VMEM((1, HW), f32)],
    )(sims, sims_col, sims, xr, sig_arr)

    return out0.reshape(B, C, H, W), out1.reshape(B, C, H, W)
